# Initial kernel scaffold; baseline (speedup 1.0000x reference)
import jax
import jax.numpy as jnp
from jax.experimental import pallas as pl


def kernel(input, weight):
    raise NotImplementedError("write your pallas kernel here")


# speedup vs baseline: 1.5588x; 1.5588x over previous
import functools

import jax
import jax.numpy as jnp
from jax import lax
from jax.experimental import pallas as pl
from jax.experimental.pallas import tpu as pltpu
from jax.experimental.pallas import tpu_sc as plsc

EMBED_DIM = 32
B_TOTAL = 16384 * 26
NC, NS = 2, 16
NW = NC * NS
B_PER_W = B_TOTAL // NW
CHUNK = 1024
N_CHUNKS = B_PER_W // CHUNK

_mesh = plsc.VectorSubcoreMesh(core_axis_name="c", subcore_axis_name="s")


@functools.partial(
    pl.kernel,
    mesh=_mesh,
    out_type=jax.ShapeDtypeStruct((B_TOTAL, EMBED_DIM), jnp.float32),
    scratch_types=[
        pltpu.VMEM((B_PER_W,), jnp.int32),
        pltpu.VMEM((CHUNK, EMBED_DIM), jnp.float32),
        pltpu.SemaphoreType.DMA,
    ],
    compiler_params=pltpu.CompilerParams(use_tc_tiling_on_sc=False),
)
def _embedding_gather(idx_hbm, table_hbm, out_hbm, idx_v, rows_v, sem):
    wid = lax.axis_index("s") * NC + lax.axis_index("c")
    base = wid * B_PER_W
    pltpu.sync_copy(idx_hbm.at[pl.ds(base, B_PER_W)], idx_v)
    for c in range(N_CHUNKS):
        pltpu.async_copy(
            table_hbm.at[idx_v.at[pl.ds(c * CHUNK, CHUNK)]], rows_v, sem
        ).wait()
        pltpu.sync_copy(rows_v, out_hbm.at[pl.ds(base + c * CHUNK, CHUNK)])


def kernel(input, weight):
    idx_flat = input.reshape(-1)
    out = _embedding_gather(idx_flat, weight)
    return out.reshape(input.shape[0], input.shape[1], EMBED_DIM)


# speedup vs baseline: 1.5723x; 1.0086x over previous
import functools

import jax
import jax.numpy as jnp
from jax import lax
from jax.experimental import pallas as pl
from jax.experimental.pallas import tpu as pltpu
from jax.experimental.pallas import tpu_sc as plsc

EMBED_DIM = 32
B_TOTAL = 16384 * 26
NC, NS = 2, 16
NW = NC * NS
B_PER_W = B_TOTAL // NW
CHUNK = 832
N_CHUNKS = B_PER_W // CHUNK
NBUF = 4
LOOKAHEAD = 2

_mesh = plsc.VectorSubcoreMesh(core_axis_name="c", subcore_axis_name="s")


@functools.partial(
    pl.kernel,
    mesh=_mesh,
    out_type=jax.ShapeDtypeStruct((B_TOTAL, EMBED_DIM), jnp.float32),
    scratch_types=[
        pltpu.VMEM((B_PER_W,), jnp.int32),
        pltpu.VMEM((NBUF, CHUNK, EMBED_DIM), jnp.float32),
        [pltpu.SemaphoreType.DMA] * NBUF,
        [pltpu.SemaphoreType.DMA] * NBUF,
    ],
    compiler_params=pltpu.CompilerParams(use_tc_tiling_on_sc=False),
)
def _embedding_gather(idx_hbm, table_hbm, out_hbm, idx_v, rows_v, sems_g, sems_w):
    wid = lax.axis_index("s") * NC + lax.axis_index("c")
    base = wid * B_PER_W
    pltpu.sync_copy(idx_hbm.at[pl.ds(base, B_PER_W)], idx_v)

    gathers = [None] * N_CHUNKS
    writes = [None] * N_CHUNKS

    def fire_gather(c):
        slot = c % NBUF
        gathers[c] = pltpu.async_copy(
            table_hbm.at[idx_v.at[pl.ds(c * CHUNK, CHUNK)]],
            rows_v.at[slot],
            sems_g[slot],
        )

    for c in range(min(LOOKAHEAD, N_CHUNKS)):
        fire_gather(c)
    for c in range(N_CHUNKS):
        nc = c + LOOKAHEAD
        if nc < N_CHUNKS:
            if nc >= NBUF:
                writes[nc - NBUF].wait()
            fire_gather(nc)
        gathers[c].wait()
        slot = c % NBUF
        writes[c] = pltpu.async_copy(
            rows_v.at[slot],
            out_hbm.at[pl.ds(base + c * CHUNK, CHUNK)],
            sems_w[slot],
        )
    for c in range(max(0, N_CHUNKS - NBUF), N_CHUNKS):
        writes[c].wait()


def kernel(input, weight):
    idx_flat = input.reshape(-1)
    out = _embedding_gather(idx_flat, weight)
    return out.reshape(input.shape[0], input.shape[1], EMBED_DIM)


# speedup vs baseline: 1.6723x; 1.0636x over previous
import functools

import jax
import jax.numpy as jnp
from jax import lax
from jax.experimental import pallas as pl
from jax.experimental.pallas import tpu as pltpu
from jax.experimental.pallas import tpu_sc as plsc

EMBED_DIM = 32
B_TOTAL = 16384 * 26
NC, NS = 2, 16
NW = NC * NS
B_PER_W = B_TOTAL // NW
CHUNK = 832
N_CHUNKS = B_PER_W // CHUNK
NBUF = 4
LOOKAHEAD = 2

_mesh = plsc.VectorSubcoreMesh(core_axis_name="c", subcore_axis_name="s")


@functools.partial(
    pl.kernel,
    mesh=_mesh,
    out_type=jax.ShapeDtypeStruct((B_TOTAL, EMBED_DIM), jnp.float32),
    scratch_types=[
        pltpu.VMEM((B_PER_W,), jnp.int32),
        pltpu.VMEM((NBUF, CHUNK, EMBED_DIM), jnp.float32),
        [pltpu.SemaphoreType.DMA] * NBUF,
        [pltpu.SemaphoreType.DMA] * NBUF,
    ],
    compiler_params=pltpu.CompilerParams(use_tc_tiling_on_sc=False),
)
def _embedding_gather(idx_hbm, table_hbm, out_hbm, idx_v, rows_v, sems_g, sems_w):
    wid = lax.axis_index("s") * NC + lax.axis_index("c")
    base = wid * B_PER_W
    pltpu.sync_copy(idx_hbm.at[pl.ds(base, B_PER_W)], idx_v)

    gathers = [None] * N_CHUNKS
    writes = [None] * N_CHUNKS

    def fire_gather(c):
        slot = c % NBUF
        gathers[c] = pltpu.async_copy(
            table_hbm.at[idx_v.at[pl.ds(c * CHUNK, CHUNK)]],
            rows_v.at[slot],
            sems_g[slot],
        )

    for c in range(min(LOOKAHEAD, N_CHUNKS)):
        fire_gather(c)
    for c in range(N_CHUNKS):
        nc = c + LOOKAHEAD
        if nc < N_CHUNKS:
            if nc >= NBUF:
                writes[nc - NBUF].wait()
            fire_gather(nc)
        gathers[c].wait()
        slot = c % NBUF
        writes[c] = pltpu.async_copy(
            rows_v.at[slot],
            out_hbm.at[pl.ds(base + c * CHUNK, CHUNK)],
            sems_w[slot],
        )
    for c in range(max(0, N_CHUNKS - NBUF), N_CHUNKS):
        writes[c].wait()


def kernel(input, weight):
    idx_flat = input.T.reshape(-1)
    out = _embedding_gather(idx_flat, weight)
    out3 = out.reshape(input.shape[1], input.shape[0], EMBED_DIM)
    return jnp.transpose(out3, (1, 0, 2))
